# baseline (speedup 1.0000x reference)
import math

import jax
import jax.numpy as jnp
from jax import lax
from jax.experimental import pallas as pl
from jax.experimental.pallas import tpu as pltpu
from jax.experimental.pallas import tpu_sc as plsc

DEG = 16
H = 4
D = 128
HD = H * D
K = 8
GM = 0.1
CM = 0.1
SLOPE = 0.2
NC = 2
NS = 16
NW = NC * NS
LANES = 16
GRP = 16


def _lrelu(v):
    return jnp.where(v >= 0, v, SLOPE * v)


def _bcast_lane(v, i):
    idx = jnp.full((LANES,), i, dtype=jnp.int32)
    return jnp.take_along_axis(v, idx, axis=0, mode="promise_in_bounds")


def _tc_matmul(x, w, a, n_blk):
    n = x.shape[0]
    din = x.shape[1]
    grid = n // n_blk

    def body(x_ref, w_ref, a_ref, h_ref, e_ref):
        h = jnp.dot(x_ref[...], w_ref[...], preferred_element_type=jnp.float32)
        h_ref[...] = h
        e_ref[...] = jnp.dot(h, a_ref[...], preferred_element_type=jnp.float32,
                             precision=jax.lax.Precision.HIGHEST)

    return pl.pallas_call(
        body,
        grid=(grid,),
        in_specs=[
            pl.BlockSpec((n_blk, din), lambda i: (i, 0)),
            pl.BlockSpec((din, HD), lambda i: (0, 0)),
            pl.BlockSpec((HD, 2 * H), lambda i: (0, 0)),
        ],
        out_specs=[
            pl.BlockSpec((n_blk, HD), lambda i: (i, 0)),
            pl.BlockSpec((n_blk, 2 * H), lambda i: (i, 0)),
        ],
        out_shape=[
            jax.ShapeDtypeStruct((n, HD), jnp.float32),
            jax.ShapeDtypeStruct((n, 2 * H), jnp.float32),
        ],
    )(x, w, a)


def _sc_kernel(n, ch):
    mesh = plsc.VectorSubcoreMesh(
        core_axis_name="c", subcore_axis_name="s",
        num_cores=NC, num_subcores=NS)

    def body(h_hbm, el_hbm, er_hbm, lbl_hbm, src_hbm, neg_hbm,
             out_hbm, lpart_hbm,
             el_v, er_v, lbl_v, src_v, neg_v, rows_a, rows_b, idx_a, idx_b,
             accbuf_v, lp_v, sem_a, sem_b):
        c = lax.axis_index("c")
        s = lax.axis_index("s")
        wid = c * NS + s
        n0 = wid * ch
        cnt = jnp.minimum(ch, n - n0)
        ngrp = cnt // GRP

        pltpu.sync_copy(el_hbm, el_v)
        pltpu.sync_copy(er_hbm, er_v)
        pltpu.sync_copy(lbl_hbm, lbl_v)
        pltpu.sync_copy(src_hbm.at[pl.ds(n0 * DEG, ch * DEG)], src_v)
        pltpu.sync_copy(neg_hbm.at[pl.ds(n0 * DEG, ch * DEG)], neg_v)

        lanes = lax.iota(jnp.int32, LANES)
        k_mask = lanes < K
        k_maskf = k_mask.astype(jnp.float32)

        def issue(i, idx_ref, rows_ref, sem):
            idx_ref[...] = src_v[pl.ds(i * DEG, DEG)]
            pltpu.async_copy(h_hbm.at[idx_ref], rows_ref, sem)

        def wait(rows_ref, sem):
            pltpu.make_async_copy(
                h_hbm.at[pl.ds(0, DEG)], rows_ref, sem).wait()

        def losswork(i, acc_g, acc_c):
            nn = n0 + i
            src_vec = src_v[pl.ds(i * DEG, DEG)]
            neg_vec = neg_v[pl.ds(i * DEG, DEG)]
            nvec = jnp.full((LANES,), nn, dtype=jnp.int32)
            lbl_s = plsc.load_gather(lbl_v, [src_vec])
            lbl_n = plsc.load_gather(lbl_v, [nvec])
            adj_f = (lbl_s == lbl_n).astype(jnp.float32)
            notadj_f = 1.0 - adj_f

            w_rows = []
            for hh in range(H):
                elg = plsc.load_gather(el_v, [src_vec * H + hh])
                ern = plsc.load_gather(er_v, [neg_vec * H + hh])
                ero = plsc.load_gather(er_v, [nvec * H + hh])
                e_h = _lrelu(elg + ero)
                en_h = _lrelu(elg + ern)

                for ii in range(DEG):
                    e_i = _bcast_lane(e_h, ii)
                    acc_g = acc_g + jnp.maximum(en_h + GM - e_i, 0.0)
                    t = jnp.maximum(e_h + CM - e_i, 0.0) * notadj_f
                    acc_c = acc_c + t * _bcast_lane(adj_f, ii)

                sk, sv = plsc.sort_key_val(e_h, lanes, descending=True)
                m = _bcast_lane(sk, 0)
                aexp = jnp.exp(sk - m) * k_maskf
                aw = aexp / jnp.sum(aexp)
                _, w_deg = plsc.sort_key_val(sv, aw)
                w_rows.append(w_deg)
            return acc_g, acc_c, w_rows

        def aggregate(j, w_rows, rows_ref):
            for hh in range(H):
                acc = [jnp.zeros((LANES,), jnp.float32)
                       for _ in range(D // LANES)]
                for d in range(DEG):
                    wb = _bcast_lane(w_rows[hh], d)
                    for v in range(D // LANES):
                        acc[v] = acc[v] + wb * rows_ref[
                            d, pl.ds(hh * D + v * LANES, LANES)]
                for v in range(D // LANES):
                    accbuf_v[pl.ds(j * HD + hh * D + v * LANES, LANES)] = acc[v]

        def pair_body(g, jj, acc_g, acc_c):
            a = g * GRP + 2 * jj
            b = a + 1
            issue(b, idx_b, rows_b, sem_b)
            acc_g, acc_c, w_a = losswork(a, acc_g, acc_c)
            wait(rows_a, sem_a)
            aggregate(2 * jj, w_a, rows_a)
            nxt = a + 2

            @pl.when(nxt < cnt)
            def _():
                issue(nxt, idx_a, rows_a, sem_a)

            acc_g, acc_c, w_b = losswork(b, acc_g, acc_c)
            wait(rows_b, sem_b)
            aggregate(2 * jj + 1, w_b, rows_b)
            return acc_g, acc_c

        def grp_body(g, carry):
            acc_g, acc_c = lax.fori_loop(
                0, GRP // 2, lambda jj, cc: pair_body(g, jj, *cc), carry)
            pltpu.sync_copy(
                accbuf_v, out_hbm.at[pl.ds((n0 + g * GRP) * HD, GRP * HD)])
            return acc_g, acc_c

        zero = jnp.zeros((LANES,), jnp.float32)
        issue(0, idx_a, rows_a, sem_a)
        acc_g, acc_c = lax.fori_loop(0, ngrp, grp_body, (zero, zero))
        lp_v[pl.ds(0, LANES)] = acc_g
        lp_v[pl.ds(LANES, LANES)] = acc_c
        pltpu.sync_copy(lp_v, lpart_hbm.at[pl.ds(wid * 2 * LANES, 2 * LANES)])

    return pl.kernel(
        body,
        out_type=[
            jax.ShapeDtypeStruct((n * HD,), jnp.float32),
            jax.ShapeDtypeStruct((NW * 2 * LANES,), jnp.float32),
        ],
        mesh=mesh,
        compiler_params=pltpu.CompilerParams(needs_layout_passes=False),
        scratch_types=[
            pltpu.VMEM((n * H,), jnp.float32),
            pltpu.VMEM((n * H,), jnp.float32),
            pltpu.VMEM((n,), jnp.int32),
            pltpu.VMEM((ch * DEG,), jnp.int32),
            pltpu.VMEM((ch * DEG,), jnp.int32),
            pltpu.VMEM((DEG, HD), jnp.float32),
            pltpu.VMEM((DEG, HD), jnp.float32),
            pltpu.VMEM((DEG,), jnp.int32),
            pltpu.VMEM((DEG,), jnp.int32),
            pltpu.VMEM((GRP * HD,), jnp.float32),
            pltpu.VMEM((2 * LANES,), jnp.float32),
            pltpu.SemaphoreType.DMA,
            pltpu.SemaphoreType.DMA,
        ],
    )


def kernel(x, src, neg_dst, labels, fc_w, attn_l, attn_r):
    n, din = x.shape
    w = fc_w.T
    eye = jnp.eye(H, dtype=jnp.float32)
    a_l = (attn_l[0][:, :, None] * eye[:, None, :]).reshape(HD, H)
    a_r = (attn_r[0][:, :, None] * eye[:, None, :]).reshape(HD, H)
    a = jnp.concatenate([a_l, a_r], axis=1)

    n_blk = 1000 if n % 1000 == 0 else n
    h, e = _tc_matmul(x, w, a, n_blk)
    el = e[:, :H].reshape(-1)
    er = e[:, H:].reshape(-1)

    ch = (math.ceil(n / NW) + 7) // 8 * 8
    pad = NW * ch - n
    src_p = jnp.pad(src, ((0, pad), (0, 0))).reshape(-1)
    neg_p = jnp.pad(neg_dst, ((0, pad), (0, 0))).reshape(-1)

    rst_flat, lpart = _sc_kernel(n, ch)(
        h, el, er, labels.astype(jnp.int32), src_p, neg_p)
    lpart = lpart.reshape(NW, 2 * LANES)
    denom = float(n * H)
    graph_loss = jnp.sum(lpart[:, :LANES]) / denom
    class_loss = jnp.sum(lpart[:, LANES:]) / denom
    return rst_flat.reshape(n, H, D), graph_loss, class_loss


# speedup vs baseline: 1.0205x; 1.0205x over previous
import math

import jax
import jax.numpy as jnp
from jax import lax
from jax.experimental import pallas as pl
from jax.experimental.pallas import tpu as pltpu
from jax.experimental.pallas import tpu_sc as plsc

DEG = 16
H = 4
D = 128
HD = H * D
K = 8
GM = 0.1
CM = 0.1
SLOPE = 0.2
NC = 2
NS = 16
NW = NC * NS
LANES = 16
GRP = 8


def _lrelu(v):
    return jnp.where(v >= 0, v, SLOPE * v)


def _bcast_lane(v, i):
    idx = jnp.full((LANES,), i, dtype=jnp.int32)
    return jnp.take_along_axis(v, idx, axis=0, mode="promise_in_bounds")


def _tc_matmul(x, w, a, n_blk):
    n = x.shape[0]
    din = x.shape[1]
    grid = n // n_blk

    def body(x_ref, w_ref, a_ref, h_ref, e_ref):
        h = jnp.dot(x_ref[...], w_ref[...], preferred_element_type=jnp.float32)
        h_ref[...] = h
        e_ref[...] = jnp.dot(h, a_ref[...], preferred_element_type=jnp.float32,
                             precision=jax.lax.Precision.HIGHEST)

    return pl.pallas_call(
        body,
        grid=(grid,),
        in_specs=[
            pl.BlockSpec((n_blk, din), lambda i: (i, 0)),
            pl.BlockSpec((din, HD), lambda i: (0, 0)),
            pl.BlockSpec((HD, 2 * H), lambda i: (0, 0)),
        ],
        out_specs=[
            pl.BlockSpec((n_blk, HD), lambda i: (i, 0)),
            pl.BlockSpec((n_blk, 2 * H), lambda i: (i, 0)),
        ],
        out_shape=[
            jax.ShapeDtypeStruct((n, HD), jnp.float32),
            jax.ShapeDtypeStruct((n, 2 * H), jnp.float32),
        ],
    )(x, w, a)


def _sc_kernel(n, ch):
    mesh = plsc.VectorSubcoreMesh(
        core_axis_name="c", subcore_axis_name="s",
        num_cores=NC, num_subcores=NS)

    def body(h_hbm, el_hbm, er_hbm, lbl_hbm, src_hbm, neg_hbm,
             out_hbm, lpart_hbm,
             el_v, er_v, lbl_v, src_v, neg_v, rows_a, rows_b, idx_a, idx_b,
             accbuf_v, lp_v, sem_a, sem_b):
        c = lax.axis_index("c")
        s = lax.axis_index("s")
        wid = c * NS + s
        n0 = wid * ch
        cnt = jnp.minimum(ch, n - n0)
        ngrp = cnt // GRP

        pltpu.sync_copy(el_hbm, el_v)
        pltpu.sync_copy(er_hbm, er_v)
        pltpu.sync_copy(lbl_hbm, lbl_v)
        pltpu.sync_copy(src_hbm.at[pl.ds(n0 * DEG, ch * DEG)], src_v)
        pltpu.sync_copy(neg_hbm.at[pl.ds(n0 * DEG, ch * DEG)], neg_v)

        lanes = lax.iota(jnp.int32, LANES)
        k_mask = lanes < K
        k_maskf = k_mask.astype(jnp.float32)

        def issue(i, idx_ref, rows_ref, sem):
            idx_ref[...] = src_v[pl.ds(i * DEG, DEG)]
            pltpu.async_copy(h_hbm.at[idx_ref], rows_ref, sem)

        def wait(rows_ref, sem):
            pltpu.make_async_copy(
                h_hbm.at[pl.ds(0, DEG)], rows_ref, sem).wait()

        def losswork(i, acc_g, acc_c):
            nn = n0 + i
            src_vec = src_v[pl.ds(i * DEG, DEG)]
            neg_vec = neg_v[pl.ds(i * DEG, DEG)]
            nvec = jnp.full((LANES,), nn, dtype=jnp.int32)
            lbl_s = plsc.load_gather(lbl_v, [src_vec])
            lbl_n = plsc.load_gather(lbl_v, [nvec])
            adj_f = (lbl_s == lbl_n).astype(jnp.float32)
            notadj_f = 1.0 - adj_f

            w_rows = []
            for hh in range(H):
                elg = plsc.load_gather(el_v, [src_vec * H + hh])
                ern = plsc.load_gather(er_v, [neg_vec * H + hh])
                ero = plsc.load_gather(er_v, [nvec * H + hh])
                e_h = _lrelu(elg + ero)
                en_h = _lrelu(elg + ern)

                for ii in range(DEG):
                    e_i = _bcast_lane(e_h, ii)
                    acc_g = acc_g + jnp.maximum(en_h + GM - e_i, 0.0)
                    t = jnp.maximum(e_h + CM - e_i, 0.0) * notadj_f
                    acc_c = acc_c + t * _bcast_lane(adj_f, ii)

                sk, sv = plsc.sort_key_val(e_h, lanes, descending=True)
                m = _bcast_lane(sk, 0)
                aexp = jnp.exp(sk - m) * k_maskf
                aw = aexp / jnp.sum(aexp)
                _, w_deg = plsc.sort_key_val(sv, aw)
                w_rows.append(w_deg)
            return acc_g, acc_c, w_rows

        def aggregate(j, w_rows, rows_ref):
            for hh in range(H):
                acc = [jnp.zeros((LANES,), jnp.float32)
                       for _ in range(D // LANES)]
                for d in range(DEG):
                    wb = _bcast_lane(w_rows[hh], d)
                    for v in range(D // LANES):
                        acc[v] = acc[v] + wb * rows_ref[
                            d, pl.ds(hh * D + v * LANES, LANES)]
                for v in range(D // LANES):
                    accbuf_v[pl.ds(j * HD + hh * D + v * LANES, LANES)] = acc[v]

        def pair_body(g, jj, acc_g, acc_c):
            a = g * GRP + 2 * jj
            b = a + 1
            issue(b, idx_b, rows_b, sem_b)
            acc_g, acc_c, w_a = losswork(a, acc_g, acc_c)
            wait(rows_a, sem_a)
            aggregate(2 * jj, w_a, rows_a)
            nxt = a + 2

            @pl.when(nxt < cnt)
            def _():
                issue(nxt, idx_a, rows_a, sem_a)

            acc_g, acc_c, w_b = losswork(b, acc_g, acc_c)
            wait(rows_b, sem_b)
            aggregate(2 * jj + 1, w_b, rows_b)
            return acc_g, acc_c

        def grp_body(g, carry):
            acc_g, acc_c = lax.fori_loop(
                0, GRP // 2, lambda jj, cc: pair_body(g, jj, *cc), carry)
            pltpu.sync_copy(
                accbuf_v, out_hbm.at[pl.ds((n0 + g * GRP) * HD, GRP * HD)])
            return acc_g, acc_c

        zero = jnp.zeros((LANES,), jnp.float32)
        issue(0, idx_a, rows_a, sem_a)
        acc_g, acc_c = lax.fori_loop(0, ngrp, grp_body, (zero, zero))
        lp_v[pl.ds(0, LANES)] = acc_g
        lp_v[pl.ds(LANES, LANES)] = acc_c
        pltpu.sync_copy(lp_v, lpart_hbm.at[pl.ds(wid * 2 * LANES, 2 * LANES)])

    return pl.kernel(
        body,
        out_type=[
            jax.ShapeDtypeStruct((n * HD,), jnp.float32),
            jax.ShapeDtypeStruct((NW * 2 * LANES,), jnp.float32),
        ],
        mesh=mesh,
        compiler_params=pltpu.CompilerParams(needs_layout_passes=False),
        scratch_types=[
            pltpu.VMEM((n * H,), jnp.float32),
            pltpu.VMEM((n * H,), jnp.float32),
            pltpu.VMEM((n,), jnp.int32),
            pltpu.VMEM((ch * DEG,), jnp.int32),
            pltpu.VMEM((ch * DEG,), jnp.int32),
            pltpu.VMEM((DEG, HD), jnp.float32),
            pltpu.VMEM((DEG, HD), jnp.float32),
            pltpu.VMEM((DEG,), jnp.int32),
            pltpu.VMEM((DEG,), jnp.int32),
            pltpu.VMEM((GRP * HD,), jnp.float32),
            pltpu.VMEM((2 * LANES,), jnp.float32),
            pltpu.SemaphoreType.DMA,
            pltpu.SemaphoreType.DMA,
        ],
    )


def kernel(x, src, neg_dst, labels, fc_w, attn_l, attn_r):
    n, din = x.shape
    w = fc_w.T
    eye = jnp.eye(H, dtype=jnp.float32)
    a_l = (attn_l[0][:, :, None] * eye[:, None, :]).reshape(HD, H)
    a_r = (attn_r[0][:, :, None] * eye[:, None, :]).reshape(HD, H)
    a = jnp.concatenate([a_l, a_r], axis=1)

    n_blk = 1000 if n % 1000 == 0 else n
    h, e = _tc_matmul(x, w, a, n_blk)
    el = e[:, :H].reshape(-1)
    er = e[:, H:].reshape(-1)

    ch = (math.ceil(n / NW) + 7) // 8 * 8
    pad = NW * ch - n
    src_p = jnp.pad(src, ((0, pad), (0, 0))).reshape(-1)
    neg_p = jnp.pad(neg_dst, ((0, pad), (0, 0))).reshape(-1)

    rst_flat, lpart = _sc_kernel(n, ch)(
        h, el, er, labels.astype(jnp.int32), src_p, neg_p)
    lpart = lpart.reshape(NW, 2 * LANES)
    denom = float(n * H)
    graph_loss = jnp.sum(lpart[:, :LANES]) / denom
    class_loss = jnp.sum(lpart[:, LANES:]) / denom
    return rst_flat.reshape(n, H, D), graph_loss, class_loss
